# baseline (speedup 1.0000x reference)
import functools

import jax
import jax.numpy as jnp
from jax import lax
from jax.experimental import pallas as pl
from jax.experimental.pallas import tpu as pltpu
from jax.experimental.pallas import tpu_sc as plsc

NC = 2
NS = 16
NW = NC * NS
CHUNK = 128


def _sc_aggregate(h, packed, zeros, n, acc_rows, chunks_per_tile):
    d = h.shape[1]
    cpt = chunks_per_tile
    zstripe = acc_rows // NS
    ostripe = (n // NS) & ~7
    tail = n - NS * ostripe
    mesh = plsc.VectorSubcoreMesh(core_axis_name="c", subcore_axis_name="s")

    @functools.partial(
        pl.kernel,
        out_type=jax.ShapeDtypeStruct((2 * n, d), jnp.float32),
        mesh=mesh,
        scratch_types=[
            pltpu.VMEM((cpt, CHUNK), jnp.int32),
            pltpu.VMEM((CHUNK,), jnp.int32),
            pltpu.VMEM((CHUNK,), jnp.int32),
            pltpu.VMEM((CHUNK,), jnp.int32),
            pltpu.VMEM((CHUNK,), jnp.int32),
            pltpu.VMEM((CHUNK, d), jnp.float32),
            pltpu.VMEM((CHUNK, d), jnp.float32),
            pltpu.VMEM_SHARED((acc_rows, d), jnp.float32),
            pltpu.SemaphoreType.DMA,
            pltpu.SemaphoreType.DMA,
        ],
    )
    def k(h_hbm, pk_hbm, z_hbm, out_hbm,
          pk_all, src0, src1, dst0, dst1, rows0, rows1, acc, sem0, sem1):
        c = lax.axis_index("c")
        s = lax.axis_index("s")
        wid = s * NC + c

        def unpack(i, src_v, dst_v):
            for j in range(CHUNK // 16):
                v = pk_all[i, pl.ds(16 * j, 16)]
                src_v[pl.ds(16 * j, 16)] = lax.bitwise_and(v, 0xFFFF)
                dst_v[pl.ds(16 * j, 16)] = lax.shift_right_logical(v, 16)

        pltpu.sync_copy(pk_hbm.at[wid], pk_all)
        unpack(0, src0, dst0)
        pltpu.async_copy(h_hbm.at[src0], rows0, sem0)
        pltpu.sync_copy(z_hbm.at[pl.ds(s * zstripe, zstripe)],
                        acc.at[pl.ds(s * zstripe, zstripe)])
        plsc.subcore_barrier()

        @pl.loop(0, cpt // 2)
        def _(g):
            i0 = 2 * g
            unpack(i0 + 1, src1, dst1)
            pltpu.async_copy(h_hbm.at[src1], rows1, sem1)
            pltpu.make_async_copy(h_hbm.at[src0], rows0, sem0).wait()
            pltpu.sync_copy(rows0, acc.at[dst0], add=True)

            @pl.when(g < cpt // 2 - 1)
            def _():
                unpack(i0 + 2, src0, dst0)
                pltpu.async_copy(h_hbm.at[src0], rows0, sem0)

            pltpu.make_async_copy(h_hbm.at[src1], rows1, sem1).wait()
            pltpu.sync_copy(rows1, acc.at[dst1], add=True)

        plsc.subcore_barrier()
        pltpu.sync_copy(acc.at[pl.ds(s * ostripe, ostripe)],
                        out_hbm.at[pl.ds(c * n + s * ostripe, ostripe)])
        if tail:
            @pl.when(s == NS - 1)
            def _():
                pltpu.sync_copy(acc.at[pl.ds(NS * ostripe, tail)],
                                out_hbm.at[pl.ds(c * n + NS * ostripe, tail)])

    return k(h, packed, zeros)


def _tc_prep(edge_index, n, acc_rows, e_pad, d):
    e = edge_index.shape[1]
    rows = e_pad // CHUNK
    grid = 16
    brows = rows // grid
    be = brows * CHUNK
    zrows = acc_rows // grid

    def body(e_ref, o_ref, z_ref):
        i = pl.program_id(0)
        r = jax.lax.broadcasted_iota(jnp.int32, (brows, CHUNK), 0)
        cidx = jax.lax.broadcasted_iota(jnp.int32, (brows, CHUNK), 1)
        ids = i * be + r * CHUNK + cidx
        src_raw = jnp.reshape(e_ref[0, :], (brows, CHUNK))
        dst_raw = jnp.reshape(e_ref[1, :], (brows, CHUNK))
        real = ids < e
        src = jnp.where(real, src_raw, jnp.bitwise_and(ids, 8191))
        dst = jnp.where(real, dst_raw, n + jnp.bitwise_and(ids, 63))
        o_ref[...] = jnp.bitwise_or(src, jnp.left_shift(dst, 16))
        z_ref[...] = jnp.zeros((zrows, d), jnp.float32)

    return pl.pallas_call(
        body,
        grid=(grid,),
        in_specs=[pl.BlockSpec((2, be), lambda i: (0, i))],
        out_specs=[pl.BlockSpec((brows, CHUNK), lambda i: (i, 0)),
                   pl.BlockSpec((zrows, d), lambda i: (i, 0))],
        out_shape=[jax.ShapeDtypeStruct((rows, CHUNK), jnp.int32),
                   jax.ShapeDtypeStruct((acc_rows, d), jnp.float32)],
    )(edge_index)


def _mm3(a, w):
    ah = a.astype(jnp.bfloat16)
    al = (a - ah.astype(jnp.float32)).astype(jnp.bfloat16)
    wh = w.astype(jnp.bfloat16)
    wl = (w - wh.astype(jnp.float32)).astype(jnp.bfloat16)
    f = jnp.float32
    return (jnp.dot(ah, wh, preferred_element_type=f)
            + jnp.dot(ah, wl, preferred_element_type=f)
            + jnp.dot(al, wh, preferred_element_type=f))


def _tc_root(h, wo_t, b2d, blk):
    n, d = h.shape

    def body(h_ref, wo_ref, b_ref, o_ref):
        o_ref[...] = _mm3(h_ref[...], wo_ref[...]) + b_ref[...]

    return pl.pallas_call(
        body,
        grid=(n // blk,),
        in_specs=[
            pl.BlockSpec((blk, d), lambda i: (i, 0)),
            pl.BlockSpec((d, d), lambda i: (0, 0)),
            pl.BlockSpec((1, d), lambda i: (0, 0)),
        ],
        out_specs=pl.BlockSpec((blk, d), lambda i: (i, 0)),
        out_shape=jax.ShapeDtypeStruct((n, d), jnp.float32),
    )(h, wo_t, b2d)


def _tc_combine(parts, root, wr_t, relu, blk):
    n2, d = parts.shape
    n = n2 // 2
    nb = n // blk

    def body(p0_ref, p1_ref, r_ref, wr_ref, o_ref):
        agg = p0_ref[...] + p1_ref[...]
        out = _mm3(agg, wr_ref[...]) + r_ref[...]
        if relu:
            out = jnp.maximum(out, 0.0)
        o_ref[...] = out

    return pl.pallas_call(
        body,
        grid=(nb,),
        in_specs=[
            pl.BlockSpec((blk, d), lambda i: (i, 0)),
            pl.BlockSpec((blk, d), lambda i, _nb=nb: (i + _nb, 0)),
            pl.BlockSpec((blk, d), lambda i: (i, 0)),
            pl.BlockSpec((d, d), lambda i: (0, 0)),
        ],
        out_specs=pl.BlockSpec((blk, d), lambda i: (i, 0)),
        out_shape=jax.ShapeDtypeStruct((n, d), jnp.float32),
    )(parts, parts, root, wr_t)


def kernel(x, edge_index, W_rel1, W_root1, b1, W_rel2, W_root2, b2,
           W_rel3, W_root3, b3):
    n, d = x.shape
    e = edge_index.shape[1]

    tile_span = NW * CHUNK
    chunks_per_tile = -(-e // tile_span)
    chunks_per_tile += chunks_per_tile % 2
    e_pad = chunks_per_tile * tile_span
    acc_rows = -(-(n + 1) // (NS * 8)) * (NS * 8)

    packed, zeros = _tc_prep(edge_index.astype(jnp.int32), n, acc_rows,
                             e_pad, d)
    packed = packed.reshape(NW, chunks_per_tile, CHUNK)

    blk = 1000 if n % 1000 == 0 else 8
    layers = [
        (W_rel1, W_root1, b1, True),
        (W_rel2, W_root2, b2, True),
        (W_rel3, W_root3, b3, False),
    ]
    h = x
    for wr, wo, b, relu in layers:
        parts = _sc_aggregate(h, packed, zeros, n, acc_rows, chunks_per_tile)
        root = _tc_root(h, wo.T, b.reshape(1, d), blk)
        h = _tc_combine(parts, root, wr.T, relu, blk)
    return h


# speedup vs baseline: 1.0103x; 1.0103x over previous
import functools

import jax
import jax.numpy as jnp
from jax import lax
from jax.experimental import pallas as pl
from jax.experimental.pallas import tpu as pltpu
from jax.experimental.pallas import tpu_sc as plsc

NC = 2
NS = 16
NW = NC * NS
CHUNK = 128


def _sc_aggregate(h, packed, zeros, n, acc_rows, chunks_per_tile):
    d = h.shape[1]
    cpt = chunks_per_tile
    zstripe = acc_rows // NS
    ostripe = (n // NS) & ~7
    tail = n - NS * ostripe
    mesh = plsc.VectorSubcoreMesh(core_axis_name="c", subcore_axis_name="s")

    @functools.partial(
        pl.kernel,
        out_type=jax.ShapeDtypeStruct((2 * n, d), jnp.float32),
        mesh=mesh,
        scratch_types=[
            pltpu.VMEM((cpt, CHUNK), jnp.int32),
            pltpu.VMEM((CHUNK,), jnp.int32),
            pltpu.VMEM((CHUNK,), jnp.int32),
            pltpu.VMEM((CHUNK,), jnp.int32),
            pltpu.VMEM((CHUNK,), jnp.int32),
            pltpu.VMEM((CHUNK, d), jnp.float32),
            pltpu.VMEM((CHUNK, d), jnp.float32),
            pltpu.VMEM_SHARED((acc_rows, d), jnp.float32),
            pltpu.SemaphoreType.DMA,
            pltpu.SemaphoreType.DMA,
        ],
    )
    def k(h_hbm, pk_hbm, z_hbm, out_hbm,
          pk_all, src0, src1, dst0, dst1, rows0, rows1, acc, sem0, sem1):
        c = lax.axis_index("c")
        s = lax.axis_index("s")
        wid = s * NC + c

        def unpack(i, src_v, dst_v):
            for j in range(CHUNK // 16):
                v = pk_all[i, pl.ds(16 * j, 16)]
                src_v[pl.ds(16 * j, 16)] = lax.bitwise_and(v, 0xFFFF)
                dst_v[pl.ds(16 * j, 16)] = lax.shift_right_logical(v, 16)

        pltpu.sync_copy(pk_hbm.at[wid], pk_all)
        unpack(0, src0, dst0)
        pltpu.async_copy(h_hbm.at[src0], rows0, sem0)
        pltpu.sync_copy(z_hbm.at[pl.ds(s * zstripe, zstripe)],
                        acc.at[pl.ds(s * zstripe, zstripe)])
        plsc.subcore_barrier()

        @pl.loop(0, cpt // 2)
        def _(g):
            i0 = 2 * g
            unpack(i0 + 1, src1, dst1)
            pltpu.async_copy(h_hbm.at[src1], rows1, sem1)
            pltpu.make_async_copy(h_hbm.at[src0], rows0, sem0).wait()
            pltpu.sync_copy(rows0, acc.at[dst0], add=True)

            @pl.when(g < cpt // 2 - 1)
            def _():
                unpack(i0 + 2, src0, dst0)
                pltpu.async_copy(h_hbm.at[src0], rows0, sem0)

            pltpu.make_async_copy(h_hbm.at[src1], rows1, sem1).wait()
            pltpu.sync_copy(rows1, acc.at[dst1], add=True)

        plsc.subcore_barrier()
        pltpu.sync_copy(acc.at[pl.ds(s * ostripe, ostripe)],
                        out_hbm.at[pl.ds(c * n + s * ostripe, ostripe)])
        if tail:
            @pl.when(s == NS - 1)
            def _():
                pltpu.sync_copy(acc.at[pl.ds(NS * ostripe, tail)],
                                out_hbm.at[pl.ds(c * n + NS * ostripe, tail)])

    return k(h, packed, zeros)


def _tc_prep(edge_index, n, acc_rows, e_pad, d):
    e = edge_index.shape[1]
    rows = e_pad // CHUNK
    grid = 16
    brows = rows // grid
    be = brows * CHUNK
    zrows = acc_rows // grid

    def body(e_ref, o_ref, z_ref):
        i = pl.program_id(0)
        r = jax.lax.broadcasted_iota(jnp.int32, (brows, CHUNK), 0)
        cidx = jax.lax.broadcasted_iota(jnp.int32, (brows, CHUNK), 1)
        ids = i * be + r * CHUNK + cidx
        src_raw = jnp.reshape(e_ref[0, :], (brows, CHUNK))
        dst_raw = jnp.reshape(e_ref[1, :], (brows, CHUNK))
        real = ids < e
        src = jnp.where(real, src_raw, jnp.bitwise_and(ids, 8191))
        dst = jnp.where(real, dst_raw, n + jnp.bitwise_and(ids, 63))
        o_ref[...] = jnp.bitwise_or(src, jnp.left_shift(dst, 16))
        z_ref[...] = jnp.zeros((zrows, d), jnp.float32)

    return pl.pallas_call(
        body,
        grid=(grid,),
        in_specs=[pl.BlockSpec((2, be), lambda i: (0, i))],
        out_specs=[pl.BlockSpec((brows, CHUNK), lambda i: (i, 0)),
                   pl.BlockSpec((zrows, d), lambda i: (i, 0))],
        out_shape=[jax.ShapeDtypeStruct((rows, CHUNK), jnp.int32),
                   jax.ShapeDtypeStruct((acc_rows, d), jnp.float32)],
    )(edge_index)


def _mm3(a, w):
    return jnp.dot(a, w, preferred_element_type=jnp.float32)


def _tc_root(h, wo_t, b2d, blk):
    n, d = h.shape

    def body(h_ref, wo_ref, b_ref, o_ref):
        o_ref[...] = _mm3(h_ref[...], wo_ref[...]) + b_ref[...]

    return pl.pallas_call(
        body,
        grid=(n // blk,),
        in_specs=[
            pl.BlockSpec((blk, d), lambda i: (i, 0)),
            pl.BlockSpec((d, d), lambda i: (0, 0)),
            pl.BlockSpec((1, d), lambda i: (0, 0)),
        ],
        out_specs=pl.BlockSpec((blk, d), lambda i: (i, 0)),
        out_shape=jax.ShapeDtypeStruct((n, d), jnp.float32),
    )(h, wo_t, b2d)


def _tc_combine(parts, root, wr_t, relu, blk):
    n2, d = parts.shape
    n = n2 // 2
    nb = n // blk

    def body(p0_ref, p1_ref, r_ref, wr_ref, o_ref):
        agg = p0_ref[...] + p1_ref[...]
        out = _mm3(agg, wr_ref[...]) + r_ref[...]
        if relu:
            out = jnp.maximum(out, 0.0)
        o_ref[...] = out

    return pl.pallas_call(
        body,
        grid=(nb,),
        in_specs=[
            pl.BlockSpec((blk, d), lambda i: (i, 0)),
            pl.BlockSpec((blk, d), lambda i, _nb=nb: (i + _nb, 0)),
            pl.BlockSpec((blk, d), lambda i: (i, 0)),
            pl.BlockSpec((d, d), lambda i: (0, 0)),
        ],
        out_specs=pl.BlockSpec((blk, d), lambda i: (i, 0)),
        out_shape=jax.ShapeDtypeStruct((n, d), jnp.float32),
    )(parts, parts, root, wr_t)


def kernel(x, edge_index, W_rel1, W_root1, b1, W_rel2, W_root2, b2,
           W_rel3, W_root3, b3):
    n, d = x.shape
    e = edge_index.shape[1]

    tile_span = NW * CHUNK
    chunks_per_tile = -(-e // tile_span)
    chunks_per_tile += chunks_per_tile % 2
    e_pad = chunks_per_tile * tile_span
    acc_rows = -(-(n + 1) // (NS * 8)) * (NS * 8)

    packed, zeros = _tc_prep(edge_index.astype(jnp.int32), n, acc_rows,
                             e_pad, d)
    packed = packed.reshape(NW, chunks_per_tile, CHUNK)

    blk = 1000 if n % 1000 == 0 else 8
    layers = [
        (W_rel1, W_root1, b1, True),
        (W_rel2, W_root2, b2, True),
        (W_rel3, W_root3, b3, False),
    ]
    h = x
    for wr, wo, b, relu in layers:
        parts = _sc_aggregate(h, packed, zeros, n, acc_rows, chunks_per_tile)
        root = _tc_root(h, wo.T, b.reshape(1, d), blk)
        h = _tc_combine(parts, root, wr.T, relu, blk)
    return h


# speedup vs baseline: 1.0286x; 1.0181x over previous
import functools

import jax
import jax.numpy as jnp
from jax import lax
from jax.experimental import pallas as pl
from jax.experimental.pallas import tpu as pltpu
from jax.experimental.pallas import tpu_sc as plsc

NC = 2
NS = 16
NW = NC * NS
CHUNK = 128


def _sc_aggregate(h, packed, zeros, n, acc_rows, chunks_per_tile):
    d = h.shape[1]
    cpt = chunks_per_tile
    zstripe = acc_rows // NS
    ostripe = (n // NS) & ~7
    tail = n - NS * ostripe
    mesh = plsc.VectorSubcoreMesh(core_axis_name="c", subcore_axis_name="s")

    @functools.partial(
        pl.kernel,
        out_type=jax.ShapeDtypeStruct((2 * n, d), jnp.float32),
        mesh=mesh,
        scratch_types=[
            pltpu.VMEM((cpt, CHUNK), jnp.int32),
            pltpu.VMEM((CHUNK,), jnp.int32),
            pltpu.VMEM((CHUNK,), jnp.int32),
            pltpu.VMEM((CHUNK,), jnp.int32),
            pltpu.VMEM((CHUNK,), jnp.int32),
            pltpu.VMEM((CHUNK, d), jnp.float32),
            pltpu.VMEM((CHUNK, d), jnp.float32),
            pltpu.VMEM_SHARED((acc_rows, d), jnp.float32),
            pltpu.SemaphoreType.DMA,
            pltpu.SemaphoreType.DMA,
        ],
    )
    def k(h_hbm, pk_hbm, z_hbm, out_hbm,
          pk_all, src0, src1, dst0, dst1, rows0, rows1, acc, sem0, sem1):
        c = lax.axis_index("c")
        s = lax.axis_index("s")
        wid = s * NC + c

        def unpack(i, src_v, dst_v):
            for j in range(CHUNK // 16):
                v = pk_all[i, pl.ds(16 * j, 16)]
                src_v[pl.ds(16 * j, 16)] = lax.bitwise_and(v, 0xFFFF)
                dst_v[pl.ds(16 * j, 16)] = lax.shift_right_logical(v, 16)

        pltpu.sync_copy(pk_hbm.at[wid], pk_all)
        unpack(0, src0, dst0)
        pltpu.async_copy(h_hbm.at[src0], rows0, sem0)
        pltpu.sync_copy(z_hbm.at[pl.ds(s * zstripe, zstripe)],
                        acc.at[pl.ds(s * zstripe, zstripe)])
        plsc.subcore_barrier()

        @pl.loop(0, cpt // 2)
        def _(g):
            i0 = 2 * g
            unpack(i0 + 1, src1, dst1)
            pltpu.async_copy(h_hbm.at[src1], rows1, sem1)
            pltpu.make_async_copy(h_hbm.at[src0], rows0, sem0).wait()
            pltpu.sync_copy(rows0, acc.at[dst0], add=True)

            @pl.when(g < cpt // 2 - 1)
            def _():
                unpack(i0 + 2, src0, dst0)
                pltpu.async_copy(h_hbm.at[src0], rows0, sem0)

            pltpu.make_async_copy(h_hbm.at[src1], rows1, sem1).wait()
            pltpu.sync_copy(rows1, acc.at[dst1], add=True)

        plsc.subcore_barrier()
        pltpu.sync_copy(acc.at[pl.ds(s * ostripe, ostripe)],
                        out_hbm.at[pl.ds(c * n + s * ostripe, ostripe)])
        if tail:
            @pl.when(s == NS - 1)
            def _():
                pltpu.sync_copy(acc.at[pl.ds(NS * ostripe, tail)],
                                out_hbm.at[pl.ds(c * n + NS * ostripe, tail)])

    return k(h, packed, zeros)


def _tc_prep(edge_index, n, acc_rows, e_pad, d):
    e = edge_index.shape[1]
    rows = e_pad // CHUNK
    grid = 16
    brows = rows // grid
    be = brows * CHUNK
    zrows = acc_rows // grid

    def body(e_ref, o_ref, z_ref):
        i = pl.program_id(0)
        r = jax.lax.broadcasted_iota(jnp.int32, (brows, CHUNK), 0)
        cidx = jax.lax.broadcasted_iota(jnp.int32, (brows, CHUNK), 1)
        ids = i * be + r * CHUNK + cidx
        src_raw = jnp.reshape(e_ref[0, :], (brows, CHUNK))
        dst_raw = jnp.reshape(e_ref[1, :], (brows, CHUNK))
        real = ids < e
        src = jnp.where(real, src_raw, jnp.bitwise_and(ids, 8191))
        dst = jnp.where(real, dst_raw, n + jnp.bitwise_and(ids, 63))
        o_ref[...] = jnp.bitwise_or(src, jnp.left_shift(dst, 16))
        z_ref[...] = jnp.zeros((zrows, d), jnp.float32)

    return pl.pallas_call(
        body,
        grid=(grid,),
        in_specs=[pl.BlockSpec((2, be), lambda i: (0, i))],
        out_specs=[pl.BlockSpec((brows, CHUNK), lambda i: (i, 0)),
                   pl.BlockSpec((zrows, d), lambda i: (i, 0))],
        out_shape=[jax.ShapeDtypeStruct((rows, CHUNK), jnp.int32),
                   jax.ShapeDtypeStruct((acc_rows, d), jnp.float32)],
    )(edge_index)


def _mm3(a, w):
    return jnp.dot(a, w, preferred_element_type=jnp.float32)


def _tc_root(h, wo_t, b2d, blk):
    n, d = h.shape

    def body(h_ref, wo_ref, b_ref, o_ref):
        o_ref[...] = _mm3(h_ref[...], wo_ref[...]) + b_ref[...]

    return pl.pallas_call(
        body,
        grid=(n // blk,),
        in_specs=[
            pl.BlockSpec((blk, d), lambda i: (i, 0)),
            pl.BlockSpec((d, d), lambda i: (0, 0)),
            pl.BlockSpec((1, d), lambda i: (0, 0)),
        ],
        out_specs=pl.BlockSpec((blk, d), lambda i: (i, 0)),
        out_shape=jax.ShapeDtypeStruct((n, d), jnp.float32),
    )(h, wo_t, b2d)


def _tc_combine(parts, root, wr_t, relu, blk):
    n2, d = parts.shape
    n = n2 // 2
    nb = n // blk

    def body(p0_ref, p1_ref, r_ref, wr_ref, o_ref):
        agg = p0_ref[...] + p1_ref[...]
        out = _mm3(agg, wr_ref[...]) + r_ref[...]
        if relu:
            out = jnp.maximum(out, 0.0)
        o_ref[...] = out

    return pl.pallas_call(
        body,
        grid=(nb,),
        in_specs=[
            pl.BlockSpec((blk, d), lambda i: (i, 0)),
            pl.BlockSpec((blk, d), lambda i, _nb=nb: (i + _nb, 0)),
            pl.BlockSpec((blk, d), lambda i: (i, 0)),
            pl.BlockSpec((d, d), lambda i: (0, 0)),
        ],
        out_specs=pl.BlockSpec((blk, d), lambda i: (i, 0)),
        out_shape=jax.ShapeDtypeStruct((n, d), jnp.float32),
    )(parts, parts, root, wr_t)


def kernel(x, edge_index, W_rel1, W_root1, b1, W_rel2, W_root2, b2,
           W_rel3, W_root3, b3):
    n, d = x.shape
    e = edge_index.shape[1]

    tile_span = NW * CHUNK
    chunks_per_tile = -(-e // tile_span)
    chunks_per_tile += chunks_per_tile % 2
    e_pad = chunks_per_tile * tile_span
    acc_rows = -(-(n + 1) // (NS * 8)) * (NS * 8)

    packed, zeros = _tc_prep(edge_index.astype(jnp.int32), n, acc_rows,
                             e_pad, d)
    packed = packed.reshape(NW, chunks_per_tile, CHUNK)

    blk = 2000 if n % 2000 == 0 else (1000 if n % 1000 == 0 else 8)
    layers = [
        (W_rel1, W_root1, b1, True),
        (W_rel2, W_root2, b2, True),
        (W_rel3, W_root3, b3, False),
    ]
    h = x
    for wr, wo, b, relu in layers:
        parts = _sc_aggregate(h, packed, zeros, n, acc_rows, chunks_per_tile)
        root = _tc_root(h, wo.T, b.reshape(1, d), blk)
        h = _tc_combine(parts, root, wr.T, relu, blk)
    return h


# speedup vs baseline: 1.0400x; 1.0111x over previous
import functools

import jax
import jax.numpy as jnp
from jax import lax
from jax.experimental import pallas as pl
from jax.experimental.pallas import tpu as pltpu
from jax.experimental.pallas import tpu_sc as plsc

NC = 2
NS = 16
NW = NC * NS
CHUNK = 128


def _sc_aggregate(h, packed, zeros, n, acc_rows, chunks_per_tile):
    d = h.shape[1]
    cpt = chunks_per_tile
    zstripe = acc_rows // NS
    ostripe = (n // NS) & ~7
    tail = n - NS * ostripe
    mesh = plsc.VectorSubcoreMesh(core_axis_name="c", subcore_axis_name="s")

    @functools.partial(
        pl.kernel,
        out_type=jax.ShapeDtypeStruct((2 * n, d), jnp.float32),
        mesh=mesh,
        scratch_types=[
            pltpu.VMEM((cpt, CHUNK), jnp.int32),
            pltpu.VMEM((CHUNK,), jnp.int32),
            pltpu.VMEM((CHUNK,), jnp.int32),
            pltpu.VMEM((CHUNK,), jnp.int32),
            pltpu.VMEM((CHUNK,), jnp.int32),
            pltpu.VMEM((CHUNK, d), jnp.float32),
            pltpu.VMEM((CHUNK, d), jnp.float32),
            pltpu.VMEM_SHARED((acc_rows, d), jnp.float32),
            pltpu.SemaphoreType.DMA,
            pltpu.SemaphoreType.DMA,
        ],
    )
    def k(h_hbm, pk_hbm, z_hbm, out_hbm,
          pk_all, src0, src1, dst0, dst1, rows0, rows1, acc, sem0, sem1):
        c = lax.axis_index("c")
        s = lax.axis_index("s")
        wid = s * NC + c

        def unpack(i, src_v, dst_v):
            for j in range(CHUNK // 16):
                v = pk_all[i, pl.ds(16 * j, 16)]
                src_v[pl.ds(16 * j, 16)] = lax.bitwise_and(v, 0xFFFF)
                dst_v[pl.ds(16 * j, 16)] = lax.shift_right_logical(v, 16)

        pltpu.sync_copy(pk_hbm.at[wid], pk_all)
        unpack(0, src0, dst0)
        pltpu.async_copy(h_hbm.at[src0], rows0, sem0)
        pltpu.sync_copy(z_hbm.at[pl.ds(s * zstripe, zstripe)],
                        acc.at[pl.ds(s * zstripe, zstripe)])
        plsc.subcore_barrier()

        @pl.loop(0, cpt // 2)
        def _(g):
            i0 = 2 * g
            unpack(i0 + 1, src1, dst1)
            pltpu.async_copy(h_hbm.at[src1], rows1, sem1)
            pltpu.make_async_copy(h_hbm.at[src0], rows0, sem0).wait()
            pltpu.sync_copy(rows0, acc.at[dst0], add=True)

            @pl.when(g < cpt // 2 - 1)
            def _():
                unpack(i0 + 2, src0, dst0)
                pltpu.async_copy(h_hbm.at[src0], rows0, sem0)

            pltpu.make_async_copy(h_hbm.at[src1], rows1, sem1).wait()
            pltpu.sync_copy(rows1, acc.at[dst1], add=True)

        plsc.subcore_barrier()
        pltpu.sync_copy(acc.at[pl.ds(s * ostripe, ostripe)],
                        out_hbm.at[pl.ds(c * n + s * ostripe, ostripe)])
        if tail:
            @pl.when(s == NS - 1)
            def _():
                pltpu.sync_copy(acc.at[pl.ds(NS * ostripe, tail)],
                                out_hbm.at[pl.ds(c * n + NS * ostripe, tail)])

    return k(h, packed, zeros)


def _tc_prep(edge_index, n, acc_rows, e_pad, d):
    e = edge_index.shape[1]
    rows = e_pad // CHUNK
    grid = 8
    brows = rows // grid
    be = brows * CHUNK
    zrows = acc_rows // grid

    def body(e_ref, o_ref, z_ref):
        i = pl.program_id(0)
        r = jax.lax.broadcasted_iota(jnp.int32, (brows, CHUNK), 0)
        cidx = jax.lax.broadcasted_iota(jnp.int32, (brows, CHUNK), 1)
        ids = i * be + r * CHUNK + cidx
        src_raw = jnp.reshape(e_ref[0, :], (brows, CHUNK))
        dst_raw = jnp.reshape(e_ref[1, :], (brows, CHUNK))
        real = ids < e
        src = jnp.where(real, src_raw, jnp.bitwise_and(ids, 8191))
        dst = jnp.where(real, dst_raw, n + jnp.bitwise_and(ids, 63))
        o_ref[...] = jnp.bitwise_or(src, jnp.left_shift(dst, 16))
        z_ref[...] = jnp.zeros((zrows, d), jnp.float32)

    return pl.pallas_call(
        body,
        grid=(grid,),
        in_specs=[pl.BlockSpec((2, be), lambda i: (0, i))],
        out_specs=[pl.BlockSpec((brows, CHUNK), lambda i: (i, 0)),
                   pl.BlockSpec((zrows, d), lambda i: (i, 0))],
        out_shape=[jax.ShapeDtypeStruct((rows, CHUNK), jnp.int32),
                   jax.ShapeDtypeStruct((acc_rows, d), jnp.float32)],
    )(edge_index)


def _mm3(a, w):
    return jnp.dot(a, w, preferred_element_type=jnp.float32)


def _tc_root(h, wo_t, b2d, blk):
    n, d = h.shape

    def body(h_ref, wo_ref, b_ref, o_ref):
        o_ref[...] = _mm3(h_ref[...], wo_ref[...]) + b_ref[...]

    return pl.pallas_call(
        body,
        grid=(n // blk,),
        in_specs=[
            pl.BlockSpec((blk, d), lambda i: (i, 0)),
            pl.BlockSpec((d, d), lambda i: (0, 0)),
            pl.BlockSpec((1, d), lambda i: (0, 0)),
        ],
        out_specs=pl.BlockSpec((blk, d), lambda i: (i, 0)),
        out_shape=jax.ShapeDtypeStruct((n, d), jnp.float32),
    )(h, wo_t, b2d)


def _tc_combine(parts, root, wr_t, relu, blk):
    n2, d = parts.shape
    n = n2 // 2
    nb = n // blk

    def body(p0_ref, p1_ref, r_ref, wr_ref, o_ref):
        agg = p0_ref[...] + p1_ref[...]
        out = _mm3(agg, wr_ref[...]) + r_ref[...]
        if relu:
            out = jnp.maximum(out, 0.0)
        o_ref[...] = out

    return pl.pallas_call(
        body,
        grid=(nb,),
        in_specs=[
            pl.BlockSpec((blk, d), lambda i: (i, 0)),
            pl.BlockSpec((blk, d), lambda i, _nb=nb: (i + _nb, 0)),
            pl.BlockSpec((blk, d), lambda i: (i, 0)),
            pl.BlockSpec((d, d), lambda i: (0, 0)),
        ],
        out_specs=pl.BlockSpec((blk, d), lambda i: (i, 0)),
        out_shape=jax.ShapeDtypeStruct((n, d), jnp.float32),
    )(parts, parts, root, wr_t)


def kernel(x, edge_index, W_rel1, W_root1, b1, W_rel2, W_root2, b2,
           W_rel3, W_root3, b3):
    n, d = x.shape
    e = edge_index.shape[1]

    tile_span = NW * CHUNK
    chunks_per_tile = -(-e // tile_span)
    chunks_per_tile += chunks_per_tile % 2
    e_pad = chunks_per_tile * tile_span
    acc_rows = -(-(n + 1) // (NS * 8)) * (NS * 8)

    packed, zeros = _tc_prep(edge_index.astype(jnp.int32), n, acc_rows,
                             e_pad, d)
    packed = packed.reshape(NW, chunks_per_tile, CHUNK)

    blk = 2000 if n % 2000 == 0 else (1000 if n % 1000 == 0 else 8)
    layers = [
        (W_rel1, W_root1, b1, True),
        (W_rel2, W_root2, b2, True),
        (W_rel3, W_root3, b3, False),
    ]
    h = x
    for wr, wo, b, relu in layers:
        parts = _sc_aggregate(h, packed, zeros, n, acc_rows, chunks_per_tile)
        root = _tc_root(h, wo.T, b.reshape(1, d), blk)
        h = _tc_combine(parts, root, wr.T, relu, blk)
    return h


# speedup vs baseline: 1.0430x; 1.0029x over previous
import functools

import jax
import jax.numpy as jnp
from jax import lax
from jax.experimental import pallas as pl
from jax.experimental.pallas import tpu as pltpu
from jax.experimental.pallas import tpu_sc as plsc

NC = 2
NS = 16
NW = NC * NS
CHUNK = 128


def _sc_aggregate(h, packed, zeros, n, acc_rows, chunks_per_tile):
    d = h.shape[1]
    cpt = chunks_per_tile
    zstripe = acc_rows // NS
    ostripe = (n // NS) & ~7
    tail = n - NS * ostripe
    mesh = plsc.VectorSubcoreMesh(core_axis_name="c", subcore_axis_name="s")

    @functools.partial(
        pl.kernel,
        out_type=jax.ShapeDtypeStruct((2 * n, d), jnp.float32),
        mesh=mesh,
        scratch_types=[
            pltpu.VMEM((cpt, CHUNK), jnp.int32),
            pltpu.VMEM((CHUNK,), jnp.int32),
            pltpu.VMEM((CHUNK,), jnp.int32),
            pltpu.VMEM((CHUNK,), jnp.int32),
            pltpu.VMEM((CHUNK,), jnp.int32),
            pltpu.VMEM((CHUNK, d), jnp.float32),
            pltpu.VMEM((CHUNK, d), jnp.float32),
            pltpu.VMEM_SHARED((acc_rows, d), jnp.float32),
            pltpu.SemaphoreType.DMA,
            pltpu.SemaphoreType.DMA,
        ],
    )
    def k(h_hbm, pk_hbm, z_hbm, out_hbm,
          pk_all, src0, src1, dst0, dst1, rows0, rows1, acc, sem0, sem1):
        c = lax.axis_index("c")
        s = lax.axis_index("s")
        wid = s * NC + c

        def unpack(i, src_v, dst_v):
            for j in range(CHUNK // 16):
                v = pk_all[i, pl.ds(16 * j, 16)]
                src_v[pl.ds(16 * j, 16)] = lax.bitwise_and(v, 0xFFFF)
                dst_v[pl.ds(16 * j, 16)] = lax.shift_right_logical(v, 16)

        pltpu.sync_copy(pk_hbm.at[wid], pk_all)
        unpack(0, src0, dst0)
        pltpu.async_copy(h_hbm.at[src0], rows0, sem0)
        pltpu.sync_copy(z_hbm.at[pl.ds(s * zstripe, zstripe)],
                        acc.at[pl.ds(s * zstripe, zstripe)])
        plsc.subcore_barrier()

        @pl.loop(0, cpt // 2)
        def _(g):
            i0 = 2 * g
            unpack(i0 + 1, src1, dst1)
            pltpu.async_copy(h_hbm.at[src1], rows1, sem1)
            pltpu.make_async_copy(h_hbm.at[src0], rows0, sem0).wait()
            pltpu.sync_copy(rows0, acc.at[dst0], add=True)

            @pl.when(g < cpt // 2 - 1)
            def _():
                unpack(i0 + 2, src0, dst0)
                pltpu.async_copy(h_hbm.at[src0], rows0, sem0)

            pltpu.make_async_copy(h_hbm.at[src1], rows1, sem1).wait()
            pltpu.sync_copy(rows1, acc.at[dst1], add=True)

        plsc.subcore_barrier()
        pltpu.sync_copy(acc.at[pl.ds(s * ostripe, ostripe)],
                        out_hbm.at[pl.ds(c * n + s * ostripe, ostripe)])
        if tail:
            @pl.when(s == NS - 1)
            def _():
                pltpu.sync_copy(acc.at[pl.ds(NS * ostripe, tail)],
                                out_hbm.at[pl.ds(c * n + NS * ostripe, tail)])

    return k(h, packed, zeros)


def _tc_prep(edge_index, n, acc_rows, e_pad, d):
    e = edge_index.shape[1]
    rows = e_pad // CHUNK
    grid = 4
    brows = rows // grid
    be = brows * CHUNK
    zrows = acc_rows // grid

    def body(e_ref, o_ref, z_ref):
        i = pl.program_id(0)
        r = jax.lax.broadcasted_iota(jnp.int32, (brows, CHUNK), 0)
        cidx = jax.lax.broadcasted_iota(jnp.int32, (brows, CHUNK), 1)
        ids = i * be + r * CHUNK + cidx
        src_raw = jnp.reshape(e_ref[0, :], (brows, CHUNK))
        dst_raw = jnp.reshape(e_ref[1, :], (brows, CHUNK))
        real = ids < e
        src = jnp.where(real, src_raw, jnp.bitwise_and(ids, 8191))
        dst = jnp.where(real, dst_raw, n + jnp.bitwise_and(ids, 63))
        o_ref[...] = jnp.bitwise_or(src, jnp.left_shift(dst, 16))
        z_ref[...] = jnp.zeros((zrows, d), jnp.float32)

    return pl.pallas_call(
        body,
        grid=(grid,),
        in_specs=[pl.BlockSpec((2, be), lambda i: (0, i))],
        out_specs=[pl.BlockSpec((brows, CHUNK), lambda i: (i, 0)),
                   pl.BlockSpec((zrows, d), lambda i: (i, 0))],
        out_shape=[jax.ShapeDtypeStruct((rows, CHUNK), jnp.int32),
                   jax.ShapeDtypeStruct((acc_rows, d), jnp.float32)],
    )(edge_index)


def _mm3(a, w):
    return jnp.dot(a, w, preferred_element_type=jnp.float32)


def _tc_root(h, wo_t, b2d, blk):
    n, d = h.shape

    def body(h_ref, wo_ref, b_ref, o_ref):
        o_ref[...] = _mm3(h_ref[...], wo_ref[...]) + b_ref[...]

    return pl.pallas_call(
        body,
        grid=(n // blk,),
        in_specs=[
            pl.BlockSpec((blk, d), lambda i: (i, 0)),
            pl.BlockSpec((d, d), lambda i: (0, 0)),
            pl.BlockSpec((1, d), lambda i: (0, 0)),
        ],
        out_specs=pl.BlockSpec((blk, d), lambda i: (i, 0)),
        out_shape=jax.ShapeDtypeStruct((n, d), jnp.float32),
    )(h, wo_t, b2d)


def _tc_combine(parts, root, wr_t, relu, blk):
    n2, d = parts.shape
    n = n2 // 2
    nb = n // blk

    def body(p0_ref, p1_ref, r_ref, wr_ref, o_ref):
        agg = p0_ref[...] + p1_ref[...]
        out = _mm3(agg, wr_ref[...]) + r_ref[...]
        if relu:
            out = jnp.maximum(out, 0.0)
        o_ref[...] = out

    return pl.pallas_call(
        body,
        grid=(nb,),
        in_specs=[
            pl.BlockSpec((blk, d), lambda i: (i, 0)),
            pl.BlockSpec((blk, d), lambda i, _nb=nb: (i + _nb, 0)),
            pl.BlockSpec((blk, d), lambda i: (i, 0)),
            pl.BlockSpec((d, d), lambda i: (0, 0)),
        ],
        out_specs=pl.BlockSpec((blk, d), lambda i: (i, 0)),
        out_shape=jax.ShapeDtypeStruct((n, d), jnp.float32),
    )(parts, parts, root, wr_t)


def kernel(x, edge_index, W_rel1, W_root1, b1, W_rel2, W_root2, b2,
           W_rel3, W_root3, b3):
    n, d = x.shape
    e = edge_index.shape[1]

    tile_span = NW * CHUNK
    chunks_per_tile = -(-e // tile_span)
    chunks_per_tile += chunks_per_tile % 2
    e_pad = chunks_per_tile * tile_span
    acc_rows = -(-(n + 1) // (NS * 8)) * (NS * 8)

    packed, zeros = _tc_prep(edge_index.astype(jnp.int32), n, acc_rows,
                             e_pad, d)
    packed = packed.reshape(NW, chunks_per_tile, CHUNK)

    blk = 2000 if n % 2000 == 0 else (1000 if n % 1000 == 0 else 8)
    layers = [
        (W_rel1, W_root1, b1, True),
        (W_rel2, W_root2, b2, True),
        (W_rel3, W_root3, b3, False),
    ]
    h = x
    for wr, wo, b, relu in layers:
        parts = _sc_aggregate(h, packed, zeros, n, acc_rows, chunks_per_tile)
        root = _tc_root(h, wo.T, b.reshape(1, d), blk)
        h = _tc_combine(parts, root, wr.T, relu, blk)
    return h
